# Initial kernel scaffold; baseline (speedup 1.0000x reference)
import jax
import jax.numpy as jnp
from jax.experimental import pallas as pl


def kernel(x, edge_index, batch, node_weight, params):
    raise NotImplementedError("write your pallas kernel here")


# speedup vs baseline: 6.5570x; 6.5570x over previous
import functools

import jax
import jax.numpy as jnp
from jax import lax
from jax.experimental import pallas as pl
from jax.experimental.pallas import tpu as pltpu
from jax.experimental.pallas import tpu_sc as plsc

N = 10000
E = 320000
F_IN = 128
DIM = 20
C = 2
G = 32
NUM_LAYERS = 3

DP = 32
NC = 2
NS = 16
NW = NC * NS
CHUNK = 128
EPT = 10112
NCH = EPT // CHUNK
NP = 10112
STRIPE = NP // NS


def _sc_agg_body(h_hbm, src_hbm, dst_hbm, zeros_hbm, out_hbm,
                 src_v, dst_v, rows_v, acc):
    cid = lax.axis_index("c")
    sid = lax.axis_index("s")
    wid = cid * NS + sid
    pltpu.sync_copy(zeros_hbm, acc.at[pl.ds(sid * STRIPE, STRIPE)])
    pltpu.sync_copy(src_hbm.at[wid], src_v)
    pltpu.sync_copy(dst_hbm.at[wid], dst_v)
    plsc.subcore_barrier()

    def body(j, carry):
        pltpu.sync_copy(h_hbm.at[src_v.at[j]], rows_v)
        pltpu.sync_copy(rows_v, acc.at[dst_v.at[j]], add=True)
        return carry

    lax.fori_loop(0, NCH, body, 0)
    plsc.subcore_barrier()
    pltpu.sync_copy(acc.at[pl.ds(sid * STRIPE, STRIPE)],
                    out_hbm.at[cid, pl.ds(sid * STRIPE, STRIPE)])


def _make_sc_agg(width):
    mesh = plsc.VectorSubcoreMesh(core_axis_name="c", subcore_axis_name="s",
                                  num_cores=NC, num_subcores=NS)
    return pl.kernel(
        _sc_agg_body,
        out_type=jax.ShapeDtypeStruct((NC, NP, width), jnp.float32),
        mesh=mesh,
        compiler_params=pltpu.CompilerParams(use_tc_tiling_on_sc=False),
        scratch_types=[
            pltpu.VMEM((NCH, CHUNK), jnp.int32),
            pltpu.VMEM((NCH, CHUNK), jnp.int32),
            pltpu.VMEM((CHUNK, width), jnp.float32),
            pltpu.VMEM_SHARED((NP, width), jnp.float32),
        ],
    )


def _bn(z, g_ref, b_ref):
    mean = jnp.mean(z, axis=0, keepdims=True)
    var = jnp.mean((z - mean) ** 2, axis=0, keepdims=True)
    return (z - mean) / jnp.sqrt(var + 1e-5) * g_ref[...] + b_ref[...]


def _mlp(z, w1_ref, b1_ref, g1_ref, bt1_ref, w2_ref, b2_ref, g2_ref, bt2_ref):
    z = jnp.dot(z, w1_ref[...], preferred_element_type=jnp.float32) + b1_ref[...]
    z = _bn(z, g1_ref, bt1_ref)
    z = jnp.dot(z, w2_ref[...], preferred_element_type=jnp.float32) + b2_ref[...]
    z = _bn(z, g2_ref, bt2_ref)
    return jnp.maximum(z, 0.0)


def _layer_body(p0_ref, p1_ref, h_ref, w1_ref, b1_ref, g1_ref, bt1_ref,
                w2_ref, b2_ref, g2_ref, bt2_ref, o_ref):
    z = p0_ref[...] + p1_ref[...] + h_ref[...]
    o_ref[...] = _mlp(z, w1_ref, b1_ref, g1_ref, bt1_ref,
                      w2_ref, b2_ref, g2_ref, bt2_ref)


def _final_body(p0_ref, p1_ref, h_ref, w1_ref, b1_ref, g1_ref, bt1_ref,
                w2_ref, b2_ref, g2_ref, bt2_ref,
                nw_ref, batch_ref, fcw_ref, fcb_ref,
                emb_ref, ge_ref, lg_ref):
    z = p0_ref[...] + p1_ref[...] + h_ref[...]
    h = _mlp(z, w1_ref, b1_ref, g1_ref, bt1_ref,
             w2_ref, b2_ref, g2_ref, bt2_ref)
    emb_ref[...] = h
    gids = lax.broadcasted_iota(jnp.int32, (N, G), 1)
    mask = (batch_ref[...] == gids).astype(jnp.float32)
    wg = mask * nw_ref[...]
    ge = lax.dot_general(wg, h, (((0,), (0,)), ((), ())),
                         preferred_element_type=jnp.float32,
                         precision=lax.Precision.HIGHEST)
    ge_ref[...] = ge
    lg_ref[...] = jnp.dot(ge, fcw_ref[...],
                          preferred_element_type=jnp.float32) + fcb_ref[...]


def _pad2(a, rows, cols):
    return jnp.zeros((rows, cols), jnp.float32).at[:a.shape[0], :a.shape[1]].set(a)


def _pad_row(v, cols):
    return jnp.zeros((1, cols), jnp.float32).at[0, :v.shape[0]].set(v)


def kernel(x, edge_index, batch, node_weight, params):
    src = jnp.zeros((NW * EPT,), jnp.int32).at[:E].set(edge_index[0])
    dst = jnp.full((NW * EPT,), N, jnp.int32).at[:E].set(edge_index[1])
    src_t = src.reshape(NW, NCH, CHUNK)
    dst_t = dst.reshape(NW, NCH, CHUNK)
    zeros_wide = jnp.zeros((STRIPE, F_IN), jnp.float32)
    zeros_nar = jnp.zeros((STRIPE, DP), jnp.float32)

    lp = [params["layer%d" % i] for i in range(NUM_LAYERS)]
    w1 = [_pad2(lp[0]["W1"], F_IN, DP)] + \
         [_pad2(lp[i]["W1"], DP, DP) for i in range(1, NUM_LAYERS)]
    w2 = [_pad2(lp[i]["W2"], DP, DP) for i in range(NUM_LAYERS)]
    b1 = [_pad_row(lp[i]["b1"], DP) for i in range(NUM_LAYERS)]
    g1 = [_pad_row(lp[i]["g1"], DP) for i in range(NUM_LAYERS)]
    bt1 = [_pad_row(lp[i]["bt1"], DP) for i in range(NUM_LAYERS)]
    b2 = [_pad_row(lp[i]["b2"], DP) for i in range(NUM_LAYERS)]
    g2 = [_pad_row(lp[i]["g2"], DP) for i in range(NUM_LAYERS)]
    bt2 = [_pad_row(lp[i]["bt2"], DP) for i in range(NUM_LAYERS)]
    fcw = _pad2(params["fc_W"], DP, 128)
    fcb = _pad_row(params["fc_b"], 128)

    sc_agg_wide = _make_sc_agg(F_IN)
    sc_agg_nar = _make_sc_agg(DP)

    h = x
    for i in range(NUM_LAYERS):
        if i == 0:
            parts = sc_agg_wide(h, src_t, dst_t, zeros_wide)
        else:
            parts = sc_agg_nar(h, src_t, dst_t, zeros_nar)
        p0 = parts[0, :N]
        p1 = parts[1, :N]
        args = (p0, p1, h, w1[i], b1[i], g1[i], bt1[i],
                w2[i], b2[i], g2[i], bt2[i])
        if i < NUM_LAYERS - 1:
            h = pl.pallas_call(
                _layer_body,
                out_shape=jax.ShapeDtypeStruct((N, DP), jnp.float32),
            )(*args)
        else:
            emb, ge, lg = pl.pallas_call(
                _final_body,
                out_shape=[
                    jax.ShapeDtypeStruct((N, DP), jnp.float32),
                    jax.ShapeDtypeStruct((G, DP), jnp.float32),
                    jax.ShapeDtypeStruct((G, 128), jnp.float32),
                ],
            )(*args, node_weight.reshape(N, 1), batch.reshape(N, 1), fcw, fcb)

    node_emb = emb[:, :DIM]
    graph_emb = ge[:, :DIM]
    logits = lg[:, :C]
    return node_emb, graph_emb, logits
